# baseline (speedup 1.0000x reference)
import dataclasses
import functools

import jax
import jax.numpy as jnp
from jax import lax
from jax.experimental import pallas as pl
from jax.experimental.pallas import tpu as pltpu
from jax.experimental.pallas import tpu_sc as plsc

N = 320000
D = 128
K = 1000
KP = 1024
NC = 2
NS = 16
NW = NC * NS
BLK = 80
NB = 5

TCB = 512
SCROWS = 217600
TCROWS = N - SCROWS
TCNB = TCROWS // TCB
TCOFF = SCROWS // TCB

CHUNK = SCROWS // NW
NBLK = CHUNK // BLK
NOUT = NBLK // NB
CNTBLK = N // NW // BLK
ZR = KP // NS


def _sc_segment_sums(embeddings, lab_sc, lab_cnt):
  mesh = plsc.VectorSubcoreMesh(core_axis_name="c", subcore_axis_name="s")
  cparams = dataclasses.replace(pltpu.CompilerParams(),
                                needs_layout_passes=False)

  @functools.partial(
      pl.kernel,
      out_type=[
          jax.ShapeDtypeStruct((NC, KP, D), jnp.float32),
          jax.ShapeDtypeStruct((NC, KP), jnp.float32),
      ],
      mesh=mesh,
      compiler_params=cparams,
      scratch_types=(
          [
              pltpu.VMEM((NB, BLK, D), jnp.float32),
              pltpu.VMEM((NBLK, BLK), jnp.int32),
              pltpu.VMEM((CNTBLK, BLK), jnp.int32),
              pltpu.VMEM((KP,), jnp.float32),
              pltpu.VMEM((ZR, D), jnp.float32),
              pltpu.VMEM((NS, ZR), jnp.float32),
              pltpu.VMEM((ZR,), jnp.float32),
              pltpu.VMEM_SHARED((KP, D), jnp.float32),
              pltpu.VMEM_SHARED((NS, KP), jnp.float32),
          ]
          + [pltpu.SemaphoreType.DMA] * (2 * NB)
      ),
  )
  def kern(emb_hbm, labs_hbm, labc_hbm, sums_hbm, cnts_hbm, rows_v, lab_v,
           labc_v, cnt_v, zrow_v, red_v, cout_v, ssums, scnt_s, *sems):
    lsem = sems[:NB]
    ssem = sems[NB:]
    ci = lax.axis_index("c")
    si = lax.axis_index("s")
    wid = ci * NS + si
    base = wid * CHUNK

    zero16 = jnp.zeros((16,), jnp.float32)
    one16 = jnp.full((16,), 1.0, jnp.float32)

    @pl.loop(0, ZR)
    def _(r):
      @pl.loop(0, D, step=16)
      def _(cc):
        zrow_v[r, pl.ds(cc, 16)] = zero16

    @pl.loop(0, KP, step=16)
    def _(r):
      cnt_v[pl.ds(r, 16)] = zero16

    pltpu.sync_copy(zrow_v, ssums.at[pl.ds(si * ZR, ZR)])
    plsc.subcore_barrier()

    pltpu.sync_copy(labs_hbm.at[wid], lab_v)
    pltpu.sync_copy(labc_hbm.at[wid], labc_v)

    for b in range(NB):
      st = base + b * BLK
      pltpu.async_copy(emb_hbm.at[pl.ds(st, BLK)], rows_v.at[b], lsem[b])

    @pl.loop(0, NOUT)
    def _(o):
      for b in range(NB):
        blk = o * NB + b
        cur = base + blk * BLK
        pltpu.make_async_copy(emb_hbm.at[pl.ds(cur, BLK)], rows_v.at[b],
                              lsem[b]).wait()
        pltpu.async_copy(rows_v.at[b], ssums.at[lab_v.at[blk]], ssem[b],
                         add=True)
        for g in range(BLK // 16):
          idx = lab_v[blk, pl.ds(g * 16, 16)]
          plsc.addupdate_scatter(cnt_v, [idx], one16)

      @pl.when(o < NOUT - 1)
      def _():
        for b in range(NB):
          nxt = base + ((o + 1) * NB + b) * BLK
          pltpu.make_async_copy(rows_v.at[b], ssums.at[lab_v.at[o * NB + b]],
                                ssem[b]).wait()
          pltpu.async_copy(emb_hbm.at[pl.ds(nxt, BLK)], rows_v.at[b],
                           lsem[b])

    for b in range(NB):
      pltpu.make_async_copy(rows_v.at[b],
                            ssums.at[lab_v.at[(NOUT - 1) * NB + b]],
                            ssem[b]).wait()

    @pl.loop(0, CNTBLK - NBLK)
    def _(r):
      for g in range(BLK // 16):
        idx = labc_v[NBLK + r, pl.ds(g * 16, 16)]
        plsc.addupdate_scatter(cnt_v, [idx], one16)

    pltpu.sync_copy(cnt_v, scnt_s.at[si])
    plsc.subcore_barrier()
    for r in range(NS):
      pltpu.sync_copy(scnt_s.at[r, pl.ds(si * ZR, ZR)], red_v.at[r])
    for c in range(0, ZR, 16):
      acc = zero16
      for r in range(NS):
        acc = acc + red_v[r, pl.ds(c, 16)]
      cout_v[pl.ds(c, 16)] = acc
    pltpu.sync_copy(cout_v, cnts_hbm.at[ci, pl.ds(si * ZR, ZR)])

    pltpu.sync_copy(ssums.at[pl.ds(si * ZR, ZR)],
                    sums_hbm.at[ci, pl.ds(si * ZR, ZR)])

  return kern(embeddings, lab_sc, lab_cnt)


def _tc_partial(embeddings, lab_tc):
  def body(lab_ref, emb_ref, o_ref, acc_ref):
    i = pl.program_id(0)
    lb = lab_ref[0, 0, :]
    iot = lax.broadcasted_iota(jnp.int32, (KP, TCB), 0)
    oht = (iot == lb[None, :]).astype(jnp.bfloat16)
    x = emb_ref[...]
    hi = x.astype(jnp.bfloat16)
    lo = (x - hi.astype(jnp.float32)).astype(jnp.bfloat16)
    p = (jnp.dot(oht, hi, preferred_element_type=jnp.float32) +
         jnp.dot(oht, lo, preferred_element_type=jnp.float32))

    @pl.when(i == 0)
    def _():
      acc_ref[...] = p

    @pl.when(i > 0)
    def _():
      acc_ref[...] += p

    @pl.when(i == TCNB - 1)
    def _():
      o_ref[...] = acc_ref[...]

  return pl.pallas_call(
      body,
      grid=(TCNB,),
      in_specs=[
          pl.BlockSpec((1, 1, TCB), lambda i: (i + TCOFF, 0, 0)),
          pl.BlockSpec((TCB, D), lambda i: (i + TCOFF, 0)),
      ],
      out_specs=pl.BlockSpec((KP, D), lambda i: (0, 0)),
      out_shape=jax.ShapeDtypeStruct((KP, D), jnp.float32),
      scratch_shapes=[pltpu.VMEM((KP, D), jnp.float32)],
  )(lab_tc, embeddings)


def _tc_finalize(psums, tcsum, pcnts):
  def body(s_ref, t_ref, c_ref, o_ref):
    s = s_ref[0] + s_ref[1] + t_ref[...]
    cnt = c_ref[0] + c_ref[1]
    pm = (cnt > 0.0).astype(jnp.float32)
    npres = jnp.sum(pm)
    means = s / jnp.maximum(cnt, 1.0)
    overall = jnp.sum(means * pm, axis=0, keepdims=True) / npres
    diff = (means - overall) * pm
    var = jnp.sum(diff * diff, axis=0, keepdims=True) / (npres - 1.0)
    o_ref[...] = jnp.broadcast_to(-jnp.mean(var), (1, 1))

  return pl.pallas_call(
      body,
      out_shape=jax.ShapeDtypeStruct((1, 1), jnp.float32),
  )(psums, tcsum, pcnts)


def kernel(embeddings, labels):
  labels = labels.astype(jnp.int32)
  lab_sc = labels[:SCROWS].reshape(NW, NBLK, BLK)
  lab_cnt = labels.reshape(NW, CNTBLK, BLK)
  lab_tc = labels.reshape(N // TCB, 1, TCB)
  psums, pcnts = _sc_segment_sums(embeddings, lab_sc, lab_cnt)
  tcsum = _tc_partial(embeddings, lab_tc)
  return _tc_finalize(psums, tcsum, pcnts.reshape(NC, KP, 1))[0, 0]


# speedup vs baseline: 1.5294x; 1.5294x over previous
import dataclasses
import functools

import jax
import jax.numpy as jnp
from jax import lax
from jax.experimental import pallas as pl
from jax.experimental.pallas import tpu as pltpu
from jax.experimental.pallas import tpu_sc as plsc

N = 320000
D = 128
K = 1000
KP = 1024
NC = 2
NS = 16
NW = NC * NS
CHUNK = N // NW
BLK = 80
NB = 5
NBLK = CHUNK // BLK
NOUT = CHUNK // (BLK * NB)
ZR = KP // NS


def _sc_segment_sums(embeddings, labels):
  mesh = plsc.VectorSubcoreMesh(core_axis_name="c", subcore_axis_name="s")
  cparams = dataclasses.replace(pltpu.CompilerParams(),
                                needs_layout_passes=False)

  @functools.partial(
      pl.kernel,
      out_type=[
          jax.ShapeDtypeStruct((NC, KP, D), jnp.float32),
          jax.ShapeDtypeStruct((NC, KP), jnp.float32),
      ],
      mesh=mesh,
      compiler_params=cparams,
      scratch_types=(
          [
              pltpu.VMEM((NB, BLK, D), jnp.float32),
              pltpu.VMEM((NBLK, BLK), jnp.int32),
              pltpu.VMEM((KP,), jnp.float32),
              pltpu.VMEM((ZR, D), jnp.float32),
              pltpu.VMEM((NS, ZR), jnp.float32),
              pltpu.VMEM((ZR,), jnp.float32),
              pltpu.VMEM_SHARED((KP, D), jnp.float32),
              pltpu.VMEM_SHARED((NS, KP), jnp.float32),
          ]
          + [pltpu.SemaphoreType.DMA] * (2 * NB)
      ),
  )
  def kern(emb_hbm, lab_hbm, sums_hbm, cnts_hbm, rows_v, lab_v, cnt_v,
           zrow_v, red_v, cout_v, ssums, scnt_s, *sems):
    lsem = sems[:NB]
    ssem = sems[NB:]
    ci = lax.axis_index("c")
    si = lax.axis_index("s")
    wid = ci * NS + si
    base = wid * CHUNK

    zero16 = jnp.zeros((16,), jnp.float32)
    one16 = jnp.full((16,), 1.0, jnp.float32)

    @pl.loop(0, ZR)
    def _(r):
      @pl.loop(0, D, step=16)
      def _(cc):
        zrow_v[r, pl.ds(cc, 16)] = zero16

    @pl.loop(0, KP, step=16)
    def _(r):
      cnt_v[pl.ds(r, 16)] = zero16

    pltpu.sync_copy(zrow_v, ssums.at[pl.ds(si * ZR, ZR)])
    plsc.subcore_barrier()

    pltpu.sync_copy(lab_hbm.at[wid], lab_v)

    for b in range(NB):
      st = base + b * BLK
      pltpu.async_copy(emb_hbm.at[pl.ds(st, BLK)], rows_v.at[b], lsem[b])

    @pl.loop(0, NOUT)
    def _(o):
      for b in range(NB):
        blk = o * NB + b
        cur = base + blk * BLK
        pltpu.make_async_copy(emb_hbm.at[pl.ds(cur, BLK)], rows_v.at[b],
                              lsem[b]).wait()
        pltpu.async_copy(rows_v.at[b], ssums.at[lab_v.at[blk]], ssem[b],
                         add=True)
      @pl.when(o < NOUT - 1)
      def _():
        for b in range(NB):
          nxt = base + ((o + 1) * NB + b) * BLK
          pltpu.make_async_copy(rows_v.at[b], ssums.at[lab_v.at[o * NB + b]],
                                ssem[b]).wait()
          pltpu.async_copy(emb_hbm.at[pl.ds(nxt, BLK)], rows_v.at[b],
                           lsem[b])

    for b in range(NB):
      pltpu.make_async_copy(rows_v.at[b],
                            ssums.at[lab_v.at[(NOUT - 1) * NB + b]],
                            ssem[b]).wait()

    @pl.loop(0, NBLK)
    def _(blk):
      for g in range(BLK // 16):
        idx = lab_v[blk, pl.ds(g * 16, 16)]
        plsc.addupdate_scatter(cnt_v, [idx], one16)

    pltpu.sync_copy(cnt_v, scnt_s.at[si])
    plsc.subcore_barrier()
    for r in range(NS):
      pltpu.sync_copy(scnt_s.at[r, pl.ds(si * ZR, ZR)], red_v.at[r])
    for c in range(0, ZR, 16):
      acc = zero16
      for r in range(NS):
        acc = acc + red_v[r, pl.ds(c, 16)]
      cout_v[pl.ds(c, 16)] = acc
    pltpu.sync_copy(cout_v, cnts_hbm.at[ci, pl.ds(si * ZR, ZR)])

    pltpu.sync_copy(ssums.at[pl.ds(si * ZR, ZR)],
                    sums_hbm.at[ci, pl.ds(si * ZR, ZR)])

  return kern(embeddings, labels)


def _tc_finalize(psums, pcnts):
  def body(s_ref, c_ref, o_ref):
    s = s_ref[0] + s_ref[1]
    cnt = c_ref[0] + c_ref[1]
    pm = (cnt > 0.0).astype(jnp.float32)
    npres = jnp.sum(pm)
    means = s / jnp.maximum(cnt, 1.0)
    overall = jnp.sum(means * pm, axis=0, keepdims=True) / npres
    diff = (means - overall) * pm
    var = jnp.sum(diff * diff, axis=0, keepdims=True) / (npres - 1.0)
    o_ref[...] = jnp.broadcast_to(-jnp.mean(var), (1, 1))

  return pl.pallas_call(
      body,
      out_shape=jax.ShapeDtypeStruct((1, 1), jnp.float32),
  )(psums, pcnts)


def kernel(embeddings, labels):
  labels = labels.astype(jnp.int32).reshape(NW, NBLK, BLK)
  psums, pcnts = _sc_segment_sums(embeddings, labels)
  return _tc_finalize(psums, pcnts.reshape(NC, KP, 1))[0, 0]


# speedup vs baseline: 1.6835x; 1.1008x over previous
import dataclasses
import functools

import jax
import jax.numpy as jnp
from jax import lax
from jax.experimental import pallas as pl
from jax.experimental.pallas import tpu as pltpu
from jax.experimental.pallas import tpu_sc as plsc

N = 320000
D = 128
K = 1000
KP = 1024
NC = 2
NS = 16
NW = NC * NS
CHUNK = N // NW
BLK = 80
NB = 5
NBLK = CHUNK // BLK
NOUT = CHUNK // (BLK * NB)
ZR = KP // NS


def _sc_segment_sums(embeddings, labels):
  mesh = plsc.VectorSubcoreMesh(core_axis_name="c", subcore_axis_name="s")
  cparams = dataclasses.replace(pltpu.CompilerParams(),
                                needs_layout_passes=False)

  @functools.partial(
      pl.kernel,
      out_type=[
          jax.ShapeDtypeStruct((NC, KP, D), jnp.float32),
          jax.ShapeDtypeStruct((NC, KP), jnp.float32),
      ],
      mesh=mesh,
      compiler_params=cparams,
      scratch_types=(
          [
              pltpu.VMEM((NB, BLK, D), jnp.float32),
              pltpu.VMEM((CHUNK,), jnp.int32),
              pltpu.VMEM((KP,), jnp.float32),
              pltpu.VMEM((ZR, D), jnp.float32),
              pltpu.VMEM((NS, ZR), jnp.float32),
              pltpu.VMEM((ZR,), jnp.float32),
              pltpu.VMEM_SHARED((KP, D), jnp.float32),
              pltpu.VMEM_SHARED((NS, KP), jnp.float32),
          ]
          + [pltpu.SemaphoreType.DMA] * (2 * NB + 1)
      ),
  )
  def kern(emb_hbm, lab_hbm, sums_hbm, cnts_hbm, rows_v, lab_v, cnt_v,
           zrow_v, red_v, cout_v, ssums, scnt_s, *sems):
    lsem = sems[:NB]
    ssem = sems[NB:2 * NB]
    labsem = sems[2 * NB]
    ci = lax.axis_index("c")
    si = lax.axis_index("s")
    wid = ci * NS + si
    base = wid * CHUNK

    pltpu.async_copy(lab_hbm.at[pl.ds(base, CHUNK)], lab_v, labsem)

    zero16 = jnp.zeros((16,), jnp.float32)
    one16 = jnp.full((16,), 1.0, jnp.float32)

    @pl.loop(0, ZR)
    def _(r):
      @pl.loop(0, D, step=16)
      def _(cc):
        zrow_v[r, pl.ds(cc, 16)] = zero16

    @pl.loop(0, KP, step=16)
    def _(r):
      cnt_v[pl.ds(r, 16)] = zero16

    pltpu.sync_copy(zrow_v, ssums.at[pl.ds(si * ZR, ZR)])
    plsc.subcore_barrier()

    for b in range(NB):
      st = base + b * BLK
      pltpu.async_copy(emb_hbm.at[pl.ds(st, BLK)], rows_v.at[b], lsem[b])

    pltpu.make_async_copy(lab_hbm.at[pl.ds(base, CHUNK)], lab_v,
                          labsem).wait()

    @pl.loop(0, NOUT)
    def _(o):
      for b in range(NB):
        off = (o * NB + b) * BLK
        pltpu.make_async_copy(emb_hbm.at[pl.ds(base + off, BLK)],
                              rows_v.at[b], lsem[b]).wait()
        pltpu.async_copy(rows_v.at[b], ssums.at[lab_v.at[pl.ds(off, BLK)]],
                         ssem[b], add=True)
        for g in range(BLK // 16):
          idx = lab_v[pl.ds(off + g * 16, 16)]
          plsc.addupdate_scatter(cnt_v, [idx], one16)

      @pl.when(o < NOUT - 1)
      def _():
        for b in range(NB):
          off = (o * NB + b) * BLK
          nxt = base + ((o + 1) * NB + b) * BLK
          pltpu.make_async_copy(rows_v.at[b],
                                ssums.at[lab_v.at[pl.ds(off, BLK)]],
                                ssem[b]).wait()
          pltpu.async_copy(emb_hbm.at[pl.ds(nxt, BLK)], rows_v.at[b],
                           lsem[b])

    for b in range(NB):
      off = ((NOUT - 1) * NB + b) * BLK
      pltpu.make_async_copy(rows_v.at[b],
                            ssums.at[lab_v.at[pl.ds(off, BLK)]],
                            ssem[b]).wait()

    pltpu.sync_copy(cnt_v, scnt_s.at[si])
    plsc.subcore_barrier()
    for r in range(NS):
      pltpu.sync_copy(scnt_s.at[r, pl.ds(si * ZR, ZR)], red_v.at[r])
    for c in range(0, ZR, 16):
      acc = zero16
      for r in range(NS):
        acc = acc + red_v[r, pl.ds(c, 16)]
      cout_v[pl.ds(c, 16)] = acc
    pltpu.sync_copy(cout_v, cnts_hbm.at[ci, pl.ds(si * ZR, ZR)])

    pltpu.sync_copy(ssums.at[pl.ds(si * ZR, ZR)],
                    sums_hbm.at[ci, pl.ds(si * ZR, ZR)])

  return kern(embeddings, labels)


def _tc_finalize(psums, pcnts):
  def body(s_ref, c_ref, o_ref):
    s = s_ref[0] + s_ref[1]
    cnt = c_ref[0] + c_ref[1]
    pm = (cnt > 0.0).astype(jnp.float32)
    npres = jnp.sum(pm)
    means = s / jnp.maximum(cnt, 1.0)
    overall = jnp.sum(means * pm, axis=0, keepdims=True) / npres
    diff = (means - overall) * pm
    var = jnp.sum(diff * diff, axis=0, keepdims=True) / (npres - 1.0)
    o_ref[...] = jnp.broadcast_to(-jnp.mean(var), (1, 1))

  return pl.pallas_call(
      body,
      out_shape=jax.ShapeDtypeStruct((1, 1), jnp.float32),
  )(psums, pcnts)


def kernel(embeddings, labels):
  labels = labels.astype(jnp.int32)
  psums, pcnts = _sc_segment_sums(embeddings, labels)
  return _tc_finalize(psums, pcnts.reshape(NC, KP, 1))[0, 0]
